# baseline (speedup 1.0000x reference)
import functools

import jax
import jax.numpy as jnp
from jax import lax
from jax.experimental import pallas as pl
from jax.experimental.pallas import tpu as pltpu
from jax.experimental.pallas import tpu_sc as plsc

TABLE_ROWS = 1000
TABLE_PAD = 1024
D = 128
B = 16384
NC = 2
NS = 16
NW = NC * NS
CHUNK = 128


def _mlp_body(buf_ref, w1_ref, b1_ref, w2_ref, b2_ref, out_ref, hi_ref, lo_ref):
    h = jnp.dot(buf_ref[...], w1_ref[...], preferred_element_type=jnp.float32)
    h = h + b1_ref[...]
    h = h * jax.nn.sigmoid(h)
    o = jnp.dot(h, w2_ref[...], preferred_element_type=jnp.float32)
    o = o + b2_ref[...]
    t = o * jax.nn.sigmoid(o)
    zpad = jnp.zeros((TABLE_PAD - TABLE_ROWS, D), jnp.float32)
    out_ref[0:TABLE_ROWS, :] = t
    out_ref[TABLE_ROWS:TABLE_PAD, :] = zpad
    hi = t.astype(jnp.bfloat16)
    lo = (t - hi.astype(jnp.float32)).astype(jnp.bfloat16)
    hi_ref[0:TABLE_ROWS, :] = hi
    hi_ref[TABLE_ROWS:TABLE_PAD, :] = zpad.astype(jnp.bfloat16)
    lo_ref[0:TABLE_ROWS, :] = lo
    lo_ref[TABLE_ROWS:TABLE_PAD, :] = zpad.astype(jnp.bfloat16)


def _mlp_table(buffer, W1, b1, W2, b2):
    return pl.pallas_call(
        _mlp_body,
        out_shape=(
            jax.ShapeDtypeStruct((TABLE_PAD, D), jnp.float32),
            jax.ShapeDtypeStruct((TABLE_PAD, D), jnp.bfloat16),
            jax.ShapeDtypeStruct((TABLE_PAD, D), jnp.bfloat16),
        ),
    )(buffer, W1, b1.reshape(1, D), W2, b2.reshape(1, D))


TC_BLOCK = 512
B_TC = 8192
B_SC = B - B_TC


def _tc_gather_body(idx_ref, hi_ref, lo_ref, out_ref):
    idx = idx_ref[0]
    iota = lax.broadcasted_iota(jnp.int32, (1, TABLE_PAD), 1)
    oh = (idx == iota).astype(jnp.bfloat16)
    out_ref[...] = jnp.dot(
        oh, hi_ref[...], preferred_element_type=jnp.float32
    ) + jnp.dot(oh, lo_ref[...], preferred_element_type=jnp.float32)


def _tc_gather(idx3, hi, lo):
    nblk = B_TC // TC_BLOCK
    return pl.pallas_call(
        _tc_gather_body,
        grid=(nblk,),
        in_specs=[
            pl.BlockSpec((1, TC_BLOCK, 1), lambda i: (i, 0, 0)),
            pl.BlockSpec((TABLE_PAD, D), lambda i: (0, 0)),
            pl.BlockSpec((TABLE_PAD, D), lambda i: (0, 0)),
        ],
        out_specs=pl.BlockSpec((TC_BLOCK, D), lambda i: (i, 0)),
        out_shape=jax.ShapeDtypeStruct((B_TC, D), jnp.float32),
    )(idx3, hi, lo)


B_PER_W = B_SC // NW
N_CHUNK = B_PER_W // CHUNK


@functools.lru_cache(maxsize=1)
def _make_gather():
    @functools.partial(
        pl.kernel,
        out_type=jax.ShapeDtypeStruct((B_SC, D), jnp.float32),
        scratch_types=[
            pltpu.VMEM((N_CHUNK, CHUNK), jnp.int32),
            pltpu.VMEM((B_PER_W, D), jnp.float32),
            pltpu.VMEM_SHARED((TABLE_PAD, D), jnp.float32),
        ]
        + [pltpu.SemaphoreType.DMA] * (N_CHUNK + 2),
        mesh=plsc.VectorSubcoreMesh(core_axis_name="c", subcore_axis_name="s"),
    )
    def _gather(steps_hbm, table_hbm, out_hbm, idx_v, rows_v, table_s, *sems):
        gsems, wsem, tsem = sems[:N_CHUNK], sems[N_CHUNK], sems[N_CHUNK + 1]
        sid = lax.axis_index("s")
        wid = sid * NC + lax.axis_index("c")
        rows_per_tile = TABLE_PAD // NS
        stage = pltpu.async_copy(
            table_hbm.at[pl.ds(sid * rows_per_tile, rows_per_tile)],
            table_s.at[pl.ds(sid * rows_per_tile, rows_per_tile)],
            tsem,
        )
        pltpu.sync_copy(steps_hbm.at[pl.ds(wid * N_CHUNK, N_CHUNK)], idx_v)
        stage.wait()
        plsc.subcore_barrier()
        gathers = [
            pltpu.async_copy(
                table_s.at[idx_v.at[j]],
                rows_v.at[pl.ds(j * CHUNK, CHUNK)],
                gsems[j],
            )
            for j in range(N_CHUNK)
        ]
        writes = []
        for j in range(N_CHUNK):
            gathers[j].wait()
            writes.append(
                pltpu.async_copy(
                    rows_v.at[pl.ds(j * CHUNK, CHUNK)],
                    out_hbm.at[pl.ds(wid * B_PER_W + j * CHUNK, CHUNK)],
                    wsem,
                )
            )
        for w in writes:
            w.wait()

    return _gather


def kernel(steps, buffer, W1, b1, W2, b2):
    table, hi, lo = _mlp_table(buffer, W1, b1, W2, b2)
    s32 = steps.astype(jnp.int32)
    tc_out = _tc_gather(
        s32[:B_TC].reshape(B_TC // TC_BLOCK, TC_BLOCK, 1), hi, lo
    )
    sc_out = _make_gather()(s32[B_TC:].reshape(B_SC // CHUNK, CHUNK), table)
    return jnp.concatenate([tc_out, sc_out], axis=0)


# speedup vs baseline: 1.8629x; 1.8629x over previous
import functools

import jax
import jax.numpy as jnp
from jax import lax
from jax.experimental import pallas as pl
from jax.experimental.pallas import tpu as pltpu
from jax.experimental.pallas import tpu_sc as plsc

TABLE_ROWS = 1000
TABLE_PAD = 1024
D = 128
B = 16384
NC = 2
NS = 16
NW = NC * NS
B_PER_W = B // NW
CHUNK = 64
N_CHUNK = B_PER_W // CHUNK


def _mlp_body(buf_ref, w1_ref, b1_ref, w2_ref, b2_ref, out_ref):
    h = jnp.dot(buf_ref[...], w1_ref[...], preferred_element_type=jnp.float32)
    h = h + b1_ref[...]
    h = h * jax.nn.sigmoid(h)
    o = jnp.dot(h, w2_ref[...], preferred_element_type=jnp.float32)
    o = o + b2_ref[...]
    out_ref[0:TABLE_ROWS, :] = o * jax.nn.sigmoid(o)
    out_ref[TABLE_ROWS:TABLE_PAD, :] = jnp.zeros(
        (TABLE_PAD - TABLE_ROWS, D), jnp.float32
    )


def _mlp_table(buffer, W1, b1, W2, b2):
    return pl.pallas_call(
        _mlp_body,
        out_shape=jax.ShapeDtypeStruct((TABLE_PAD, D), jnp.float32),
    )(buffer, W1, b1.reshape(1, D), W2, b2.reshape(1, D))


@functools.lru_cache(maxsize=1)
def _make_gather():
    @functools.partial(
        pl.kernel,
        out_type=jax.ShapeDtypeStruct((B, D), jnp.float32),
        scratch_types=[
            pltpu.VMEM((N_CHUNK, CHUNK), jnp.int32),
            pltpu.VMEM((B_PER_W, D), jnp.float32),
            pltpu.VMEM_SHARED((TABLE_PAD, D), jnp.float32),
        ]
        + [pltpu.SemaphoreType.DMA] * (N_CHUNK + 2),
        mesh=plsc.VectorSubcoreMesh(core_axis_name="c", subcore_axis_name="s"),
    )
    def _gather(steps_hbm, table_hbm, out_hbm, idx_v, rows_v, table_s, *sems):
        gsems, wsem, tsem = sems[:N_CHUNK], sems[N_CHUNK], sems[N_CHUNK + 1]
        sid = lax.axis_index("s")
        wid = sid * NC + lax.axis_index("c")
        rows_per_tile = TABLE_PAD // NS
        stage = pltpu.async_copy(
            table_hbm.at[pl.ds(sid * rows_per_tile, rows_per_tile)],
            table_s.at[pl.ds(sid * rows_per_tile, rows_per_tile)],
            tsem,
        )
        pltpu.sync_copy(steps_hbm.at[pl.ds(wid * N_CHUNK, N_CHUNK)], idx_v)
        stage.wait()
        plsc.subcore_barrier()
        gathers = [
            pltpu.async_copy(
                table_s.at[idx_v.at[j]],
                rows_v.at[pl.ds(j * CHUNK, CHUNK)],
                gsems[j],
            )
            for j in range(N_CHUNK)
        ]
        writes = []
        for j in range(N_CHUNK):
            gathers[j].wait()
            writes.append(
                pltpu.async_copy(
                    rows_v.at[pl.ds(j * CHUNK, CHUNK)],
                    out_hbm.at[pl.ds(wid * B_PER_W + j * CHUNK, CHUNK)],
                    wsem,
                )
            )
        for w in writes:
            w.wait()

    return _gather


def kernel(steps, buffer, W1, b1, W2, b2):
    table = _mlp_table(buffer, W1, b1, W2, b2)
    steps2 = steps.astype(jnp.int32).reshape(B // CHUNK, CHUNK)
    return _make_gather()(steps2, table)


# speedup vs baseline: 1.9641x; 1.0543x over previous
import functools

import jax
import jax.numpy as jnp
from jax import lax
from jax.experimental import pallas as pl
from jax.experimental.pallas import tpu as pltpu
from jax.experimental.pallas import tpu_sc as plsc

TABLE_ROWS = 1000
TABLE_PAD = 1024
D = 128
B = 16384
NC = 2
NS = 16
NW = NC * NS
B_PER_W = B // NW
CHUNK = 128
N_CHUNK = B_PER_W // CHUNK


def _mlp_body(buf_ref, w1_ref, b1_ref, w2_ref, b2_ref, out_ref):
    h = jnp.dot(buf_ref[...], w1_ref[...], preferred_element_type=jnp.float32)
    h = h + b1_ref[...]
    h = h * jax.nn.sigmoid(h)
    o = jnp.dot(h, w2_ref[...], preferred_element_type=jnp.float32)
    o = o + b2_ref[...]
    out_ref[0:TABLE_ROWS, :] = o * jax.nn.sigmoid(o)
    out_ref[TABLE_ROWS:TABLE_PAD, :] = jnp.zeros(
        (TABLE_PAD - TABLE_ROWS, D), jnp.float32
    )


def _mlp_table(buffer, W1, b1, W2, b2):
    return pl.pallas_call(
        _mlp_body,
        out_shape=jax.ShapeDtypeStruct((TABLE_PAD, D), jnp.float32),
    )(buffer, W1, b1.reshape(1, D), W2, b2.reshape(1, D))


@functools.lru_cache(maxsize=1)
def _make_gather():
    @functools.partial(
        pl.kernel,
        out_type=jax.ShapeDtypeStruct((B, D), jnp.float32),
        scratch_types=[
            pltpu.VMEM((N_CHUNK, CHUNK), jnp.int32),
            pltpu.VMEM((B_PER_W, D), jnp.float32),
            pltpu.VMEM_SHARED((TABLE_PAD, D), jnp.float32),
        ]
        + [pltpu.SemaphoreType.DMA] * (N_CHUNK + 2),
        mesh=plsc.VectorSubcoreMesh(core_axis_name="c", subcore_axis_name="s"),
    )
    def _gather(steps_hbm, table_hbm, out_hbm, idx_v, rows_v, table_s, *sems):
        gsems, wsem, tsem = sems[:N_CHUNK], sems[N_CHUNK], sems[N_CHUNK + 1]
        sid = lax.axis_index("s")
        wid = sid * NC + lax.axis_index("c")
        rows_per_tile = TABLE_PAD // NS
        stage = pltpu.async_copy(
            table_hbm.at[pl.ds(sid * rows_per_tile, rows_per_tile)],
            table_s.at[pl.ds(sid * rows_per_tile, rows_per_tile)],
            tsem,
        )
        pltpu.sync_copy(steps_hbm.at[pl.ds(wid * N_CHUNK, N_CHUNK)], idx_v)
        gathers = [
            pltpu.async_copy(
                table_hbm.at[idx_v.at[0]],
                rows_v.at[pl.ds(0, CHUNK)],
                gsems[0],
            )
        ]
        stage.wait()
        plsc.subcore_barrier()
        writes = []
        for j in range(N_CHUNK):
            if j + 1 < N_CHUNK:
                gathers.append(
                    pltpu.async_copy(
                        table_s.at[idx_v.at[j + 1]],
                        rows_v.at[pl.ds((j + 1) * CHUNK, CHUNK)],
                        gsems[j + 1],
                    )
                )
            gathers[j].wait()
            writes.append(
                pltpu.async_copy(
                    rows_v.at[pl.ds(j * CHUNK, CHUNK)],
                    out_hbm.at[pl.ds(wid * B_PER_W + j * CHUNK, CHUNK)],
                    wsem,
                )
            )
        for w in writes:
            w.wait()

    return _gather


def kernel(steps, buffer, W1, b1, W2, b2):
    table = _mlp_table(buffer, W1, b1, W2, b2)
    steps2 = steps.astype(jnp.int32).reshape(B // CHUNK, CHUNK)
    return _make_gather()(steps2, table)
